# baseline (speedup 1.0000x reference)
import functools

import jax
import jax.numpy as jnp
from jax import lax
from jax.experimental import pallas as pl
from jax.experimental.pallas import tpu as pltpu
from jax.experimental.pallas import tpu_sc as plsc

N_USERS = 50000
N = 100000
E = 3200000
D = 16

NC = 2
NS = 16
NW = NC * NS
L = 128
G = 8
NBUF = 8
AHEAD = 4

NG = 98
CPT = NG * G
CR = CPT * NW
E_PAD = CR * L
TRIPLES = (NG - 2) // 3

N_PAD = 100096
ROWS_PT = N_PAD // NS


def _spmm_body(ego_hbm, colr, rowr, vexpr, zeros_hbm, out_hbm,
               colg, rowg, msg, vex, side, isem, gsem, vsem, ssem):
    cid = lax.axis_index("c")
    sid = lax.axis_index("s")
    wid = sid * NC + cid
    stripe = pl.ds(sid * ROWS_PT, ROWS_PT)

    pltpu.sync_copy(zeros_hbm.at[stripe], side.at[stripe])
    plsc.subcore_barrier()

    gbase = wid * NG
    cbase = gbase * G
    dummy = ego_hbm.at[pl.ds(0, L)]

    def load_group(gi, s):
        return (pltpu.async_copy(colr.at[gi], colg.at[s], isem),
                pltpu.async_copy(rowr.at[gi], rowg.at[s], isem))

    def drain_scatter():
        pltpu.make_async_copy(dummy, msg.at[0], ssem).wait()

    def wait_gather(k):
        pltpu.make_async_copy(dummy, msg.at[k], gsem).wait()
        pltpu.make_async_copy(dummy, vex.at[k], vsem).wait()

    def issue_gather(s, r, k, cj):
        pltpu.async_copy(ego_hbm.at[colg.at[s, r]], msg.at[k], gsem)
        pltpu.async_copy(vexpr.at[cj], vex.at[k], vsem)

    def scale_and_scatter(s, b, k):
        def scale(q, _):
            sl = pl.ds(q * 2, 2)
            msg[k, sl, :] = msg[k, sl, :] * vex[k, sl, :]
            return 0
        lax.fori_loop(0, L // 2, scale, 0)
        pltpu.async_copy(msg.at[k], side.at[rowg.at[s, b]], ssem, add=True)

    def run_group(g, s, nxt_s, prefetch, drain, tail):
        descs = load_group(g + 1, nxt_s) if prefetch else None
        for b in range(G):
            if prefetch and b == 3:
                for d in descs:
                    d.wait()
            issue = (b < 4) if tail else True
            if issue:
                if drain or b >= 4:
                    drain_scatter()
                cj = g * G + b + 4
                if b < 4:
                    issue_gather(s, b + 4, (b + 4) % NBUF, cj)
                else:
                    issue_gather(nxt_s, b - 4, (b + 4) % NBUF, cj)
            wait_gather(b)
            scale_and_scatter(s, b, b)

    for c in load_group(gbase, 0):
        c.wait()
    for k in range(AHEAD):
        issue_gather(0, k, k, cbase + k)
    run_group(gbase, 0, 1, True, False, False)

    def triple(t, _):
        g = gbase + 1 + t * 3
        for s in range(3):
            run_group(g + s, (1 + s) % 3, (2 + s) % 3, True, True, False)
        return 0
    lax.fori_loop(0, TRIPLES, triple, 0)

    run_group(gbase + NG - 1, 1, 2, False, True, True)
    for _ in range(NBUF):
        drain_scatter()
    plsc.subcore_barrier()

    pltpu.sync_copy(side.at[stripe], out_hbm.at[cid, stripe])


_spmm = functools.partial(
    pl.kernel,
    out_type=jax.ShapeDtypeStruct((NC, N_PAD, D), jnp.bfloat16),
    mesh=plsc.VectorSubcoreMesh(core_axis_name="c", subcore_axis_name="s",
                                num_cores=NC, num_subcores=NS),
    compiler_params=pltpu.CompilerParams(use_tc_tiling_on_sc=False,
                                        needs_layout_passes=False),
    scratch_types=[
        pltpu.VMEM((3, G, L), jnp.int32),
        pltpu.VMEM((3, G, L), jnp.int32),
        pltpu.VMEM((NBUF, L, D), jnp.bfloat16),
        pltpu.VMEM((NBUF, L, D), jnp.bfloat16),
        pltpu.VMEM_SHARED((N_PAD, D), jnp.bfloat16),
        pltpu.SemaphoreType.DMA,
        pltpu.SemaphoreType.DMA,
        pltpu.SemaphoreType.DMA,
        pltpu.SemaphoreType.DMA,
    ],
)(_spmm_body)


PACK = 128 // D
N8 = N // PACK
BLK8 = 512


N8P = N_PAD // PACK if False else None


def _dense_body(p0, p1, ego, Wg8, bg8, Wb8, bb8, ones8, ego_out, norm_out):
    side = p0[...].astype(jnp.float32) + p1[...].astype(jnp.float32)
    e = ego[...]
    s = side + e
    b = side * e
    x = (jnp.dot(s, Wg8[...], preferred_element_type=jnp.float32) + bg8[...]
         + jnp.dot(b, Wb8[...], preferred_element_type=jnp.float32) + bb8[...])
    x = jnp.where(x >= 0, x, 0.2 * x)
    ego_out[...] = x
    sq = jnp.dot(x * x, ones8[...], preferred_element_type=jnp.float32)
    norm_out[...] = x / jnp.maximum(jnp.sqrt(sq), 1e-12)


def _dense(p0, p1, ego, Wgc, bgc, Wbi, bbi):
    eye = jnp.eye(PACK, dtype=jnp.float32)
    Wg8 = jnp.kron(eye, Wgc)
    Wb8 = jnp.kron(eye, Wbi)
    ones8 = jnp.kron(eye, jnp.ones((D, D), jnp.float32))
    bg8 = jnp.tile(bgc, PACK).reshape(1, PACK * D)
    bb8 = jnp.tile(bbi, PACK).reshape(1, PACK * D)
    grid = (pl.cdiv(N8, BLK8),)
    node_spec = pl.BlockSpec((BLK8, PACK * D), lambda i: (i, 0))
    w_spec = pl.BlockSpec((PACK * D, PACK * D), lambda i: (0, 0))
    b_spec = pl.BlockSpec((1, PACK * D), lambda i: (0, 0))
    outs = pl.pallas_call(
        _dense_body,
        grid=grid,
        in_specs=[node_spec, node_spec, node_spec, w_spec, b_spec, w_spec,
                  b_spec, w_spec],
        out_specs=[node_spec, node_spec],
        out_shape=[jax.ShapeDtypeStruct((N8, PACK * D), jnp.float32),
                   jax.ShapeDtypeStruct((N8, PACK * D), jnp.float32)],
    )(p0.reshape(N_PAD // PACK, PACK * D)[:N8],
      p1.reshape(N_PAD // PACK, PACK * D)[:N8],
      ego.reshape(N8, PACK * D), Wg8, bg8, Wb8, bb8, ones8)
    return outs[0].reshape(N, D), outs[1].reshape(N, D)


def kernel(adj_indices, adj_values, emb, Wgc0, bgc0, Wbi0, bbi0,
           Wgc1, bgc1, Wbi1, bbi1):
    pad = E_PAD - E
    idx = jnp.pad(adj_indices.astype(jnp.int32), ((0, 0), (0, pad)))
    rowr = idx[0].reshape(CR // G, G, L)
    colr = idx[1].reshape(CR // G, G, L)
    vexpr = jnp.broadcast_to(
        jnp.pad(adj_values, (0, pad)).astype(jnp.bfloat16)[:, None],
        (E_PAD, D)).reshape(CR, L, D)
    zeros = jnp.zeros((N_PAD, D), jnp.bfloat16)

    ego = emb
    outs = [emb]
    for (Wgc, bgc, Wbi, bbi) in ((Wgc0, bgc0, Wbi0, bbi0),
                                 (Wgc1, bgc1, Wbi1, bbi1)):
        ego_bf = jnp.pad(ego.astype(jnp.bfloat16), ((0, N_PAD - N), (0, 0)))
        partials = _spmm(ego_bf, colr, rowr, vexpr, zeros)
        ego, norm = _dense(partials[0], partials[1], ego, Wgc, bgc, Wbi, bbi)
        outs.append(norm)
    all_e = jnp.concatenate(outs, axis=1)
    return (all_e[:N_USERS], all_e[N_USERS:])


# speedup vs baseline: 2.5496x; 2.5496x over previous
import functools

import jax
import jax.numpy as jnp
from jax import lax
from jax.experimental import pallas as pl
from jax.experimental.pallas import tpu as pltpu
from jax.experimental.pallas import tpu_sc as plsc

N_USERS = 50000
N = 100000
E = 3200000
D = 16

NC = 2
NS = 16
NW = NC * NS
L = 128
G = 8
NBUF = 8
AHEAD = 6

NG = 98
CPT = NG * G
CR = CPT * NW
E_PAD = CR * L
TRIPLES = (NG - 2) // 3

N_PAD = 100096
ROWS_PT = N_PAD // NS


def _spmm_body(ego_hbm, colr, rowr, valr, zeros_hbm, out_hbm,
               colg, rowg, valg, msg, side, isem, gsem, ssem):
    cid = lax.axis_index("c")
    sid = lax.axis_index("s")
    wid = sid * NC + cid
    stripe = pl.ds(sid * ROWS_PT, ROWS_PT)

    pltpu.sync_copy(zeros_hbm.at[stripe], side.at[stripe])
    plsc.subcore_barrier()

    gbase = wid * NG
    dummy = ego_hbm.at[pl.ds(0, L)]

    def load_group(gi, s):
        return (pltpu.async_copy(colr.at[gi], colg.at[s], isem),
                pltpu.async_copy(rowr.at[gi], rowg.at[s], isem),
                pltpu.async_copy(valr.at[gi], valg.at[s], isem))

    def drain_scatter():
        pltpu.make_async_copy(dummy, msg.at[0], ssem).wait()

    def wait_gather(k):
        pltpu.make_async_copy(dummy, msg.at[k], gsem).wait()

    def issue_gather(s, r, k):
        pltpu.async_copy(ego_hbm.at[colg.at[s, r]], msg.at[k], gsem)

    def scale_and_scatter(s, b, k):
        def scale(q, _):
            b16 = q * 16
            vv = valg[s, b, pl.ds(b16, 16)]
            for j in range(16):
                msg[k, b16 + j, :] = msg[k, b16 + j, :] * vv[j]
            return 0
        lax.fori_loop(0, L // 16, scale, 0)
        pltpu.async_copy(msg.at[k], side.at[rowg.at[s, b]], ssem, add=True)

    def run_group(g, s, nxt_s, prefetch, drain, tail):
        descs = load_group(g + 1, nxt_s) if prefetch else None
        for b in range(G):
            if prefetch and b == 1:
                for d in descs:
                    d.wait()
            issue = (b < G - AHEAD) if tail else True
            if issue:
                if drain or b >= NBUF - AHEAD:
                    drain_scatter()
                if b < G - AHEAD:
                    issue_gather(s, b + AHEAD, (b + AHEAD) % NBUF)
                else:
                    issue_gather(nxt_s, b - (G - AHEAD), (b + AHEAD) % NBUF)
            wait_gather(b)
            scale_and_scatter(s, b, b)

    for c in load_group(gbase, 0):
        c.wait()
    for k in range(AHEAD):
        issue_gather(0, k, k)

    run_group(gbase, 0, 1, True, False, False)

    def triple(t, _):
        g = gbase + 1 + t * 3
        for s in range(3):
            run_group(g + s, (1 + s) % 3, (2 + s) % 3, True, True, False)
        return 0
    lax.fori_loop(0, TRIPLES, triple, 0)

    run_group(gbase + NG - 1, 1, 2, False, True, True)
    for _ in range(NBUF):
        drain_scatter()
    plsc.subcore_barrier()

    pltpu.sync_copy(side.at[stripe], out_hbm.at[cid, stripe])


_spmm = functools.partial(
    pl.kernel,
    out_type=jax.ShapeDtypeStruct((NC, N_PAD, D), jnp.float32),
    mesh=plsc.VectorSubcoreMesh(core_axis_name="c", subcore_axis_name="s",
                                num_cores=NC, num_subcores=NS),
    compiler_params=pltpu.CompilerParams(use_tc_tiling_on_sc=False),
    scratch_types=[
        pltpu.VMEM((3, G, L), jnp.int32),
        pltpu.VMEM((3, G, L), jnp.int32),
        pltpu.VMEM((3, G, L), jnp.float32),
        pltpu.VMEM((NBUF, L, D), jnp.float32),
        pltpu.VMEM_SHARED((N_PAD, D), jnp.float32),
        pltpu.SemaphoreType.DMA,
        pltpu.SemaphoreType.DMA,
        pltpu.SemaphoreType.DMA,
    ],
)(_spmm_body)


PACK = 128 // D
N8 = N // PACK
BLK8 = 512


def _dense_body(p0, p1, ego, Wg8, bg8, Wb8, bb8, ones8, ego_out, norm_out):
    side = p0[...] + p1[...]
    e = ego[...]
    s = side + e
    b = side * e
    x = (jnp.dot(s, Wg8[...], preferred_element_type=jnp.float32) + bg8[...]
         + jnp.dot(b, Wb8[...], preferred_element_type=jnp.float32) + bb8[...])
    x = jnp.where(x >= 0, x, 0.2 * x)
    ego_out[...] = x
    sq = jnp.dot(x * x, ones8[...], preferred_element_type=jnp.float32)
    norm_out[...] = x / jnp.maximum(jnp.sqrt(sq), 1e-12)


def _dense(p0, p1, ego, Wgc, bgc, Wbi, bbi):
    eye = jnp.eye(PACK, dtype=jnp.float32)
    Wg8 = jnp.kron(eye, Wgc)
    Wb8 = jnp.kron(eye, Wbi)
    ones8 = jnp.kron(eye, jnp.ones((D, D), jnp.float32))
    bg8 = jnp.tile(bgc, PACK).reshape(1, PACK * D)
    bb8 = jnp.tile(bbi, PACK).reshape(1, PACK * D)
    grid = (pl.cdiv(N8, BLK8),)
    node_spec = pl.BlockSpec((BLK8, PACK * D), lambda i: (i, 0))
    w_spec = pl.BlockSpec((PACK * D, PACK * D), lambda i: (0, 0))
    b_spec = pl.BlockSpec((1, PACK * D), lambda i: (0, 0))
    outs = pl.pallas_call(
        _dense_body,
        grid=grid,
        in_specs=[node_spec, node_spec, node_spec, w_spec, b_spec, w_spec,
                  b_spec, w_spec],
        out_specs=[node_spec, node_spec],
        out_shape=[jax.ShapeDtypeStruct((N8, PACK * D), jnp.float32),
                   jax.ShapeDtypeStruct((N8, PACK * D), jnp.float32)],
    )(p0.reshape(N8, PACK * D), p1.reshape(N8, PACK * D),
      ego.reshape(N8, PACK * D), Wg8, bg8, Wb8, bb8, ones8)
    return outs[0].reshape(N, D), outs[1].reshape(N, D)


def kernel(adj_indices, adj_values, emb, Wgc0, bgc0, Wbi0, bbi0,
           Wgc1, bgc1, Wbi1, bbi1):
    pad = E_PAD - E
    idx = jnp.pad(adj_indices.astype(jnp.int32), ((0, 0), (0, pad)))
    rowr = idx[0].reshape(CR // G, G, L)
    colr = idx[1].reshape(CR // G, G, L)
    valr = jnp.pad(adj_values, (0, pad)).reshape(CR // G, G, L)
    zeros = jnp.zeros((N_PAD, D), jnp.float32)

    ego = emb
    outs = [emb]
    for (Wgc, bgc, Wbi, bbi) in ((Wgc0, bgc0, Wbi0, bbi0),
                                 (Wgc1, bgc1, Wbi1, bbi1)):
        partials = _spmm(ego, colr, rowr, valr, zeros)[:, :N, :]
        ego, norm = _dense(partials[0], partials[1], ego, Wgc, bgc, Wbi, bbi)
        outs.append(norm)
    all_e = jnp.concatenate(outs, axis=1)
    return (all_e[:N_USERS], all_e[N_USERS:])


# speedup vs baseline: 2.5569x; 1.0028x over previous
import functools

import jax
import jax.numpy as jnp
from jax import lax
from jax.experimental import pallas as pl
from jax.experimental.pallas import tpu as pltpu
from jax.experimental.pallas import tpu_sc as plsc

N_USERS = 50000
N = 100000
E = 3200000
D = 16

NC = 2
NS = 16
NW = NC * NS
L = 128
G = 8
NBUF = 8
AHEAD = 6

NG = 98
CPT = NG * G
CR = CPT * NW
E_PAD = CR * L
TRIPLES = (NG - 2) // 3

N_PAD = 100096
ROWS_PT = N_PAD // NS


def _spmm_body(ego_hbm, colr, rowr, valr, zeros_hbm, out_hbm,
               colg, rowg, valg, msg, side, isem, gsem, ssem):
    cid = lax.axis_index("c")
    sid = lax.axis_index("s")
    wid = sid * NC + cid
    stripe = pl.ds(sid * ROWS_PT, ROWS_PT)

    pltpu.sync_copy(zeros_hbm.at[stripe], side.at[stripe])
    plsc.subcore_barrier()

    gbase = wid * NG
    dummy = ego_hbm.at[pl.ds(0, L)]

    def load_group(gi, s):
        return (pltpu.async_copy(colr.at[gi], colg.at[s], isem),
                pltpu.async_copy(rowr.at[gi], rowg.at[s], isem),
                pltpu.async_copy(valr.at[gi], valg.at[s], isem))

    def drain_scatter():
        pltpu.make_async_copy(dummy, msg.at[0], ssem).wait()

    def wait_gather(k):
        pltpu.make_async_copy(dummy, msg.at[k], gsem).wait()

    def issue_gather(s, r, k):
        pltpu.async_copy(ego_hbm.at[colg.at[s, r, pl.ds(0, L // 2)]],
                         msg.at[k, pl.ds(0, L // 2)], gsem)
        pltpu.async_copy(ego_hbm.at[colg.at[s, r, pl.ds(L // 2, L // 2)]],
                         msg.at[k, pl.ds(L // 2, L // 2)], gsem)

    def scale_and_scatter(s, b, k):
        def scale(q, _):
            b16 = q * 16
            vv = valg[s, b, pl.ds(b16, 16)]
            for j in range(16):
                msg[k, b16 + j, :] = msg[k, b16 + j, :] * vv[j]
            return 0
        lax.fori_loop(0, L // 16, scale, 0)
        pltpu.async_copy(msg.at[k], side.at[rowg.at[s, b]], ssem, add=True)

    def run_group(g, s, nxt_s, prefetch, drain, tail):
        descs = load_group(g + 1, nxt_s) if prefetch else None
        for b in range(G):
            if prefetch and b == 1:
                for d in descs:
                    d.wait()
            issue = (b < G - AHEAD) if tail else True
            if issue:
                if drain or b >= NBUF - AHEAD:
                    drain_scatter()
                if b < G - AHEAD:
                    issue_gather(s, b + AHEAD, (b + AHEAD) % NBUF)
                else:
                    issue_gather(nxt_s, b - (G - AHEAD), (b + AHEAD) % NBUF)
            wait_gather(b)
            scale_and_scatter(s, b, b)

    for c in load_group(gbase, 0):
        c.wait()
    for k in range(AHEAD):
        issue_gather(0, k, k)

    run_group(gbase, 0, 1, True, False, False)

    def triple(t, _):
        g = gbase + 1 + t * 3
        for s in range(3):
            run_group(g + s, (1 + s) % 3, (2 + s) % 3, True, True, False)
        return 0
    lax.fori_loop(0, TRIPLES, triple, 0)

    run_group(gbase + NG - 1, 1, 2, False, True, True)
    for _ in range(NBUF):
        drain_scatter()
    plsc.subcore_barrier()

    pltpu.sync_copy(side.at[stripe], out_hbm.at[cid, stripe])


_spmm = functools.partial(
    pl.kernel,
    out_type=jax.ShapeDtypeStruct((NC, N_PAD, D), jnp.float32),
    mesh=plsc.VectorSubcoreMesh(core_axis_name="c", subcore_axis_name="s",
                                num_cores=NC, num_subcores=NS),
    compiler_params=pltpu.CompilerParams(use_tc_tiling_on_sc=False),
    scratch_types=[
        pltpu.VMEM((3, G, L), jnp.int32),
        pltpu.VMEM((3, G, L), jnp.int32),
        pltpu.VMEM((3, G, L), jnp.float32),
        pltpu.VMEM((NBUF, L, D), jnp.float32),
        pltpu.VMEM_SHARED((N_PAD, D), jnp.float32),
        pltpu.SemaphoreType.DMA,
        pltpu.SemaphoreType.DMA,
        pltpu.SemaphoreType.DMA,
    ],
)(_spmm_body)


PACK = 128 // D
N8 = N // PACK
BLK8 = 512


def _dense_body(p0, p1, ego, Wg8, bg8, Wb8, bb8, ones8, ego_out, norm_out):
    side = p0[...] + p1[...]
    e = ego[...]
    s = side + e
    b = side * e
    x = (jnp.dot(s, Wg8[...], preferred_element_type=jnp.float32) + bg8[...]
         + jnp.dot(b, Wb8[...], preferred_element_type=jnp.float32) + bb8[...])
    x = jnp.where(x >= 0, x, 0.2 * x)
    ego_out[...] = x
    sq = jnp.dot(x * x, ones8[...], preferred_element_type=jnp.float32)
    norm_out[...] = x / jnp.maximum(jnp.sqrt(sq), 1e-12)


def _dense(p0, p1, ego, Wgc, bgc, Wbi, bbi):
    eye = jnp.eye(PACK, dtype=jnp.float32)
    Wg8 = jnp.kron(eye, Wgc)
    Wb8 = jnp.kron(eye, Wbi)
    ones8 = jnp.kron(eye, jnp.ones((D, D), jnp.float32))
    bg8 = jnp.tile(bgc, PACK).reshape(1, PACK * D)
    bb8 = jnp.tile(bbi, PACK).reshape(1, PACK * D)
    grid = (pl.cdiv(N8, BLK8),)
    node_spec = pl.BlockSpec((BLK8, PACK * D), lambda i: (i, 0))
    w_spec = pl.BlockSpec((PACK * D, PACK * D), lambda i: (0, 0))
    b_spec = pl.BlockSpec((1, PACK * D), lambda i: (0, 0))
    outs = pl.pallas_call(
        _dense_body,
        grid=grid,
        in_specs=[node_spec, node_spec, node_spec, w_spec, b_spec, w_spec,
                  b_spec, w_spec],
        out_specs=[node_spec, node_spec],
        out_shape=[jax.ShapeDtypeStruct((N8, PACK * D), jnp.float32),
                   jax.ShapeDtypeStruct((N8, PACK * D), jnp.float32)],
    )(p0.reshape(N8, PACK * D), p1.reshape(N8, PACK * D),
      ego.reshape(N8, PACK * D), Wg8, bg8, Wb8, bb8, ones8)
    return outs[0].reshape(N, D), outs[1].reshape(N, D)


def kernel(adj_indices, adj_values, emb, Wgc0, bgc0, Wbi0, bbi0,
           Wgc1, bgc1, Wbi1, bbi1):
    pad = E_PAD - E
    idx = jnp.pad(adj_indices.astype(jnp.int32), ((0, 0), (0, pad)))
    rowr = idx[0].reshape(CR // G, G, L)
    colr = idx[1].reshape(CR // G, G, L)
    valr = jnp.pad(adj_values, (0, pad)).reshape(CR // G, G, L)
    zeros = jnp.zeros((N_PAD, D), jnp.float32)

    ego = emb
    outs = [emb]
    for (Wgc, bgc, Wbi, bbi) in ((Wgc0, bgc0, Wbi0, bbi0),
                                 (Wgc1, bgc1, Wbi1, bbi1)):
        partials = _spmm(ego, colr, rowr, valr, zeros)[:, :N, :]
        ego, norm = _dense(partials[0], partials[1], ego, Wgc, bgc, Wbi, bbi)
        outs.append(norm)
    all_e = jnp.concatenate(outs, axis=1)
    return (all_e[:N_USERS], all_e[N_USERS:])
